# Initial kernel scaffold; baseline (speedup 1.0000x reference)
import jax
import jax.numpy as jnp
from jax.experimental import pallas as pl


def kernel(x_in, d, L_indices, L_values, W_cl1, b_cl1, W_fc1, b_fc1, W_fc2, b_fc2, W_fc3, b_fc3, W_nn1, b_nn1, W_nn2, b_nn2, W_sum2, b_sum2):
    raise NotImplementedError("write your pallas kernel here")


# speedup vs baseline: 5.4936x; 5.4936x over previous
import functools

import jax
import jax.numpy as jnp
from jax import lax
from jax.experimental import pallas as pl
from jax.experimental.pallas import tpu as pltpu
from jax.experimental.pallas import tpu_sc as plsc

B = 16
V = 10000
E = 320000
CL1_F = 32
CL1_K = 4
POOL = 8
FC1Fin = CL1_F * (V // POOL)
V_PAD = 10240

NTILES = 16
IDXW = 128
ROWS_PER_TILE = 160
EPAD = NTILES * ROWS_PER_TILE * IDXW
CHUNK_ROWS = 8
CE = CHUNK_ROWS * IDXW
NCHUNK = ROWS_PER_TILE // CHUNK_ROWS
VROWS_T = V_PAD // NTILES


def _spmm3_body(x0_hbm, cols_hbm, rows_hbm, vals_hbm,
                x1_hbm, x2_hbm, x3_hbm,
                acc_sh, colsb, rowsb, valsb, gbuf, sbuf,
                accv, yv, pv, obuf, zbuf, gsem):
    cid = lax.axis_index("c")
    tid = lax.axis_index("s")
    iota = lax.iota(jnp.int32, 16)

    @pl.when(cid == 0)
    def _():
        def zinit(i, _):
            zbuf[i] = jnp.zeros((16,), jnp.float32)
            return _
        lax.fori_loop(0, VROWS_T, zinit, 0)
        pltpu.sync_copy(zbuf, acc_sh.at[pl.ds(tid * VROWS_T, VROWS_T)])
        plsc.subcore_barrier()

        for k, (y_hbm, prev_hbm, out_hbm) in enumerate(
                [(x0_hbm, None, x1_hbm),
                 (x1_hbm, x0_hbm, x2_hbm),
                 (x2_hbm, x1_hbm, x3_hbm)]):
            def chunk_body(c, _, y_hbm=y_hbm):
                base_row = tid * ROWS_PER_TILE + c * CHUNK_ROWS
                base_e = base_row * IDXW
                pltpu.sync_copy(cols_hbm.at[pl.ds(base_row, CHUNK_ROWS)], colsb)
                pltpu.sync_copy(rows_hbm.at[pl.ds(base_row, CHUNK_ROWS)], rowsb)
                pltpu.sync_copy(vals_hbm.at[pl.ds(base_e, CE)], valsb)
                descs = []
                for j in range(CHUNK_ROWS):
                    descs.append(pltpu.async_copy(
                        y_hbm.at[colsb.at[j]],
                        gbuf.at[pl.ds(j * IDXW, IDXW)], gsem))
                for d in descs:
                    d.wait()

                def scale(i, _):
                    i16 = jnp.full((16,), i, jnp.int32)
                    sv = plsc.load_gather(valsb, [i16])
                    sbuf[i] = sv * gbuf[i]
                    return _
                lax.fori_loop(0, CE, scale, 0)
                for j in range(CHUNK_ROWS):
                    pltpu.sync_copy(sbuf.at[pl.ds(j * IDXW, IDXW)],
                                    acc_sh.at[rowsb.at[j]], add=True)
                return _
            lax.fori_loop(0, NCHUNK, chunk_body, 0)
            plsc.subcore_barrier()

            r0 = tid * VROWS_T
            pltpu.sync_copy(acc_sh.at[pl.ds(r0, VROWS_T)], accv)
            pltpu.sync_copy(y_hbm.at[pl.ds(r0, VROWS_T)], yv)
            if prev_hbm is not None:
                pltpu.sync_copy(prev_hbm.at[pl.ds(r0, VROWS_T)], pv)

            def comb(i, _, first=(prev_hbm is None)):
                a = accv[i]
                y = yv[i]
                if first:
                    o = a - y
                else:
                    o = 2.0 * a - 2.0 * y - pv[i]
                obuf[i] = o
                return _
            lax.fori_loop(0, VROWS_T, comb, 0)
            pltpu.sync_copy(obuf, out_hbm.at[pl.ds(r0, VROWS_T)])
            pltpu.sync_copy(zbuf, acc_sh.at[pl.ds(r0, VROWS_T)])
            plsc.subcore_barrier()


def _spmm3(x0, cols2, rows2, vals_pad):
    kern = functools.partial(
        pl.kernel,
        mesh=plsc.VectorSubcoreMesh(core_axis_name="c", subcore_axis_name="s"),
        compiler_params=pltpu.CompilerParams(needs_layout_passes=False,
                                             use_tc_tiling_on_sc=False),
        out_type=[jax.ShapeDtypeStruct((V_PAD, B), jnp.float32)] * 3,
        scratch_types=[
            pltpu.VMEM_SHARED((V_PAD, B), jnp.float32),
            pltpu.VMEM((CHUNK_ROWS, IDXW), jnp.int32),
            pltpu.VMEM((CHUNK_ROWS, IDXW), jnp.int32),
            pltpu.VMEM((CE,), jnp.float32),
            pltpu.VMEM((CE, B), jnp.float32),
            pltpu.VMEM((CE, B), jnp.float32),
            pltpu.VMEM((VROWS_T, B), jnp.float32),
            pltpu.VMEM((VROWS_T, B), jnp.float32),
            pltpu.VMEM((VROWS_T, B), jnp.float32),
            pltpu.VMEM((VROWS_T, B), jnp.float32),
            pltpu.VMEM((VROWS_T, B), jnp.float32),
            pltpu.SemaphoreType.DMA,
        ],
    )(_spmm3_body)
    return kern(x0, cols2, rows2, vals_pad)


VB1 = 1024


def _head_body(x0_ref, x1_ref, x2_ref, x3_ref, w_ref, b_ref, out_ref):
    xcat = jnp.concatenate(
        [x0_ref[...], x1_ref[...], x2_ref[...], x3_ref[...]], axis=1)
    wt = w_ref[...].T
    eye = jnp.eye(16, dtype=jnp.float32)
    w4 = (eye[None, :, :, None] * wt[:, None, None, :]).reshape(64, 512)
    bias = jnp.tile(b_ref[...].reshape(1, 32), (1, 16))
    xg = jax.nn.relu(jnp.dot(xcat, w4, preferred_element_type=jnp.float32)
                     + bias)
    xp = xg.reshape(VB1 // POOL, POOL, 512).max(axis=1)
    xp = xp.reshape(VB1 // POOL, 16, 32)
    xp = jnp.swapaxes(xp, 1, 2).reshape(VB1 // POOL * 32, 16)
    out_ref[...] = xp


def _head(x0, x1, x2, x3, W_cl1, b_cl1):
    n = V_PAD // VB1
    return pl.pallas_call(
        _head_body,
        grid=(n,),
        in_specs=[
            pl.BlockSpec((VB1, B), lambda i: (i, 0)),
            pl.BlockSpec((VB1, B), lambda i: (i, 0)),
            pl.BlockSpec((VB1, B), lambda i: (i, 0)),
            pl.BlockSpec((VB1, B), lambda i: (i, 0)),
            pl.BlockSpec((CL1_F, CL1_K), lambda i: (0, 0)),
            pl.BlockSpec((CL1_F, 1), lambda i: (0, 0)),
        ],
        out_specs=pl.BlockSpec((VB1 // POOL * CL1_F, B), lambda i: (i, 0)),
        out_shape=jax.ShapeDtypeStruct((V_PAD // POOL * CL1_F, B), jnp.float32),
    )(x0, x1, x2, x3, W_cl1, b_cl1.reshape(CL1_F, 1))


def _nn_body(x0_ref, wnn1_ref, bnn1_ref, wnn2_ref, bnn2_ref, xn2_ref):
    xn = jax.nn.relu(jnp.dot(wnn1_ref[...], x0_ref[...],
                             preferred_element_type=jnp.float32)
                     + bnn1_ref[...])
    xn2_ref[...] = jax.nn.relu(jnp.dot(wnn2_ref[...], xn,
                                       preferred_element_type=jnp.float32)
                               + bnn2_ref[...])


def _nn(x0, W_nn1, b_nn1, W_nn2, b_nn2):
    full = lambda shape: pl.BlockSpec(shape, lambda: (0, 0))
    return pl.pallas_call(
        _nn_body,
        in_specs=[full((V, B)), full((256, V)), full((256, 1)),
                  full((128, 256)), full((128, 1))],
        out_specs=full((128, B)),
        out_shape=jax.ShapeDtypeStruct((128, B), jnp.float32),
    )(x0, W_nn1, b_nn1.reshape(256, 1), W_nn2, b_nn2.reshape(128, 1))


RB1 = 32
NR1 = 256 // RB1


def _mid_body(xpT_ref, wfc1_ref, xn2_ref,
              bfc1_ref, wfc2_ref, bfc2_ref, wsum2_ref, bsum2_ref,
              h_ref, d1T_ref, out_ref, hT_s):
    i = pl.program_id(0)
    hT_s[pl.ds(i * RB1, RB1), :] = jax.nn.relu(
        jnp.dot(wfc1_ref[...], xpT_ref[...],
                preferred_element_type=jnp.float32)
        + bfc1_ref[...])

    @pl.when(i == NR1 - 1)
    def _():
        hT = hT_s[...]
        d1 = jax.nn.relu(jnp.dot(wfc2_ref[...], hT,
                                 preferred_element_type=jnp.float32)
                         + bfc2_ref[...])
        cat = jnp.concatenate([hT, xn2_ref[...]], axis=0)
        lg = jnp.dot(wsum2_ref[...], cat,
                     preferred_element_type=jnp.float32) + bsum2_ref[...]
        m = jnp.max(lg, axis=0, keepdims=True)
        lse = jnp.log(jnp.sum(jnp.exp(lg - m), axis=0, keepdims=True)) + m
        h_ref[...] = hT.T
        d1T_ref[...] = d1
        out_ref[...] = (lg - lse).T


def _mid(xpT, xn2, W_fc1, b_fc1, W_fc2, b_fc2, W_sum2, b_sum2):
    full = lambda shape: pl.BlockSpec(shape, lambda i: (0, 0))
    return pl.pallas_call(
        _mid_body,
        grid=(NR1,),
        in_specs=[
            full((FC1Fin, B)),
            pl.BlockSpec((RB1, FC1Fin), lambda i: (i, 0)),
            full((128, B)),
            pl.BlockSpec((RB1, 1), lambda i: (i, 0)),
            full((512, 256)), full((512, 1)),
            full((2, 384)), full((2, 1)),
        ],
        out_specs=[full((B, 256)), full((512, B)), full((B, 2))],
        out_shape=[
            jax.ShapeDtypeStruct((B, 256), jnp.float32),
            jax.ShapeDtypeStruct((512, B), jnp.float32),
            jax.ShapeDtypeStruct((B, 2), jnp.float32),
        ],
        scratch_shapes=[pltpu.VMEM((256, B), jnp.float32)],
    )(xpT, W_fc1, xn2,
      b_fc1.reshape(256, 1), W_fc2, b_fc2.reshape(512, 1), W_sum2,
      b_sum2.reshape(2, 1))


MB3 = 1000


def _fc3_body(wfc3_ref, d1T_ref, b_ref, dec_ref):
    dec_ref[...] = (jnp.dot(wfc3_ref[...], d1T_ref[...],
                            preferred_element_type=jnp.float32) + b_ref[...])


def _fc3(W_fc3, d1T, b_fc3):
    n = V // MB3
    return pl.pallas_call(
        _fc3_body,
        grid=(n,),
        in_specs=[
            pl.BlockSpec((MB3, 512), lambda i: (i, 0)),
            pl.BlockSpec((512, B), lambda i: (0, 0)),
            pl.BlockSpec((MB3, 1), lambda i: (i, 0)),
        ],
        out_specs=pl.BlockSpec((MB3, B), lambda i: (i, 0)),
        out_shape=jax.ShapeDtypeStruct((V, B), jnp.float32),
    )(W_fc3, d1T, b_fc3.reshape(V, 1))


def kernel(x_in, d, L_indices, L_values, W_cl1, b_cl1, W_fc1, b_fc1,
           W_fc2, b_fc2, W_fc3, b_fc3, W_nn1, b_nn1, W_nn2, b_nn2,
           W_sum2, b_sum2):
    x0 = jnp.transpose(x_in)
    x0p = jnp.pad(x0, ((0, V_PAD - V), (0, 0)))

    npad = EPAD - E
    pad_idx = (jnp.arange(npad, dtype=jnp.int32) % V)
    rows = jnp.concatenate([L_indices[0], pad_idx])
    cols = jnp.concatenate([L_indices[1], pad_idx])
    vals = jnp.concatenate([L_values, jnp.zeros((npad,), jnp.float32)])
    rows2 = rows.reshape(EPAD // IDXW, IDXW)
    cols2 = cols.reshape(EPAD // IDXW, IDXW)

    x1, x2, x3 = _spmm3(x0p, cols2, rows2, vals)
    xpT = _head(x0p, x1, x2, x3, W_cl1, b_cl1)[:FC1Fin]
    xn2 = _nn(x0, W_nn1, b_nn1, W_nn2, b_nn2)
    h, d1T, out = _mid(xpT, xn2, W_fc1, b_fc1, W_fc2, b_fc2, W_sum2, b_sum2)
    decT = _fc3(W_fc3, d1T, b_fc3)
    return (jnp.transpose(decT), h, out)


# speedup vs baseline: 8.6936x; 1.5825x over previous
import functools

import jax
import jax.numpy as jnp
from jax import lax
from jax.experimental import pallas as pl
from jax.experimental.pallas import tpu as pltpu
from jax.experimental.pallas import tpu_sc as plsc

B = 16
V = 10000
E = 320000
CL1_F = 32
CL1_K = 4
POOL = 8
FC1Fin = CL1_F * (V // POOL)
V_PAD = 10240

NTILES = 16
IDXW = 128
ROWS_PER_TILE = 160
EPAD = NTILES * ROWS_PER_TILE * IDXW
CHUNK_ROWS = 8
CE = CHUNK_ROWS * IDXW
NCHUNK = ROWS_PER_TILE // CHUNK_ROWS
VROWS_T = V_PAD // NTILES


def _spmm3_body(x0_hbm, cols_hbm, rows_hbm, vals_hbm,
                x1_hbm, x2_hbm, x3_hbm,
                acc_sh, colsb, rowsb, valsb, gbuf, sbuf,
                accv, yv, pv, obuf, zbuf, gsem):
    cid = lax.axis_index("c")
    tid = lax.axis_index("s")
    iota = lax.iota(jnp.int32, 16)

    @pl.when(cid == 0)
    def _():
        @plsc.parallel_loop(0, VROWS_T, unroll=8)
        def _zinit(i):
            zbuf[i] = jnp.zeros((16,), jnp.float32)
        pltpu.sync_copy(zbuf, acc_sh.at[pl.ds(tid * VROWS_T, VROWS_T)])
        plsc.subcore_barrier()

        for k, (y_hbm, prev_hbm, out_hbm) in enumerate(
                [(x0_hbm, None, x1_hbm),
                 (x1_hbm, x0_hbm, x2_hbm),
                 (x2_hbm, x1_hbm, x3_hbm)]):
            def chunk_body(c, _, y_hbm=y_hbm):
                base_row = tid * ROWS_PER_TILE + c * CHUNK_ROWS
                base_e = base_row * IDXW
                pltpu.sync_copy(cols_hbm.at[pl.ds(base_row, CHUNK_ROWS)], colsb)
                pltpu.sync_copy(rows_hbm.at[pl.ds(base_row, CHUNK_ROWS)], rowsb)
                pltpu.sync_copy(vals_hbm.at[pl.ds(base_e, CE)], valsb)
                descs = []
                for j in range(CHUNK_ROWS):
                    descs.append(pltpu.async_copy(
                        y_hbm.at[colsb.at[j]],
                        gbuf.at[pl.ds(j * IDXW, IDXW)], gsem))
                for d in descs:
                    d.wait()

                @plsc.parallel_loop(0, CE, step=16, unroll=2)
                def _scale(g):
                    for j in range(16):
                        sv = plsc.load_gather(
                            valsb, [jnp.full((16,), g + j, jnp.int32)])
                        sbuf[g + j] = sv * gbuf[g + j]
                for j in range(CHUNK_ROWS):
                    pltpu.sync_copy(sbuf.at[pl.ds(j * IDXW, IDXW)],
                                    acc_sh.at[rowsb.at[j]], add=True)
                return _
            lax.fori_loop(0, NCHUNK, chunk_body, 0)
            plsc.subcore_barrier()

            r0 = tid * VROWS_T
            pltpu.sync_copy(acc_sh.at[pl.ds(r0, VROWS_T)], accv)
            pltpu.sync_copy(y_hbm.at[pl.ds(r0, VROWS_T)], yv)
            if prev_hbm is not None:
                pltpu.sync_copy(prev_hbm.at[pl.ds(r0, VROWS_T)], pv)

            first = prev_hbm is None

            @plsc.parallel_loop(0, VROWS_T, unroll=8)
            def _comb(i):
                a = accv[i]
                y = yv[i]
                if first:
                    obuf[i] = a - y
                else:
                    obuf[i] = 2.0 * a - 2.0 * y - pv[i]
            pltpu.sync_copy(obuf, out_hbm.at[pl.ds(r0, VROWS_T)])
            pltpu.sync_copy(zbuf, acc_sh.at[pl.ds(r0, VROWS_T)])
            plsc.subcore_barrier()


def _spmm3(x0, cols2, rows2, vals_pad):
    kern = functools.partial(
        pl.kernel,
        mesh=plsc.VectorSubcoreMesh(core_axis_name="c", subcore_axis_name="s"),
        compiler_params=pltpu.CompilerParams(needs_layout_passes=False,
                                             use_tc_tiling_on_sc=False),
        out_type=[jax.ShapeDtypeStruct((V_PAD, B), jnp.float32)] * 3,
        scratch_types=[
            pltpu.VMEM_SHARED((V_PAD, B), jnp.float32),
            pltpu.VMEM((CHUNK_ROWS, IDXW), jnp.int32),
            pltpu.VMEM((CHUNK_ROWS, IDXW), jnp.int32),
            pltpu.VMEM((CE,), jnp.float32),
            pltpu.VMEM((CE, B), jnp.float32),
            pltpu.VMEM((CE, B), jnp.float32),
            pltpu.VMEM((VROWS_T, B), jnp.float32),
            pltpu.VMEM((VROWS_T, B), jnp.float32),
            pltpu.VMEM((VROWS_T, B), jnp.float32),
            pltpu.VMEM((VROWS_T, B), jnp.float32),
            pltpu.VMEM((VROWS_T, B), jnp.float32),
            pltpu.SemaphoreType.DMA,
        ],
    )(_spmm3_body)
    return kern(x0, cols2, rows2, vals_pad)


VB1 = 1024


def _head_body(x0_ref, x1_ref, x2_ref, x3_ref, w_ref, b_ref, out_ref):
    xcat = jnp.concatenate(
        [x0_ref[...], x1_ref[...], x2_ref[...], x3_ref[...]], axis=1)
    wt = w_ref[...].T
    eye = jnp.eye(16, dtype=jnp.float32)
    w4 = (eye[None, :, :, None] * wt[:, None, None, :]).reshape(64, 512)
    bias = jnp.tile(b_ref[...].reshape(1, 32), (1, 16))
    xg = jax.nn.relu(jnp.dot(xcat, w4, preferred_element_type=jnp.float32)
                     + bias)
    xp = xg.reshape(VB1 // POOL, POOL, 512).max(axis=1)
    xp = xp.reshape(VB1 // POOL, 16, 32)
    xp = jnp.swapaxes(xp, 1, 2).reshape(VB1 // POOL * 32, 16)
    out_ref[...] = xp


def _head(x0, x1, x2, x3, W_cl1, b_cl1):
    n = V_PAD // VB1
    return pl.pallas_call(
        _head_body,
        grid=(n,),
        in_specs=[
            pl.BlockSpec((VB1, B), lambda i: (i, 0)),
            pl.BlockSpec((VB1, B), lambda i: (i, 0)),
            pl.BlockSpec((VB1, B), lambda i: (i, 0)),
            pl.BlockSpec((VB1, B), lambda i: (i, 0)),
            pl.BlockSpec((CL1_F, CL1_K), lambda i: (0, 0)),
            pl.BlockSpec((CL1_F, 1), lambda i: (0, 0)),
        ],
        out_specs=pl.BlockSpec((VB1 // POOL * CL1_F, B), lambda i: (i, 0)),
        out_shape=jax.ShapeDtypeStruct((V_PAD // POOL * CL1_F, B), jnp.float32),
    )(x0, x1, x2, x3, W_cl1, b_cl1.reshape(CL1_F, 1))


def _nn_body(x0_ref, wnn1_ref, bnn1_ref, wnn2_ref, bnn2_ref, xn2_ref):
    xn = jax.nn.relu(jnp.dot(wnn1_ref[...], x0_ref[...],
                             preferred_element_type=jnp.float32)
                     + bnn1_ref[...])
    xn2_ref[...] = jax.nn.relu(jnp.dot(wnn2_ref[...], xn,
                                       preferred_element_type=jnp.float32)
                               + bnn2_ref[...])


def _nn(x0, W_nn1, b_nn1, W_nn2, b_nn2):
    full = lambda shape: pl.BlockSpec(shape, lambda: (0, 0))
    return pl.pallas_call(
        _nn_body,
        in_specs=[full((V, B)), full((256, V)), full((256, 1)),
                  full((128, 256)), full((128, 1))],
        out_specs=full((128, B)),
        out_shape=jax.ShapeDtypeStruct((128, B), jnp.float32),
    )(x0, W_nn1, b_nn1.reshape(256, 1), W_nn2, b_nn2.reshape(128, 1))


RB1 = 32
NR1 = 256 // RB1


def _mid_body(xpT_ref, wfc1_ref, xn2_ref,
              bfc1_ref, wfc2_ref, bfc2_ref, wsum2_ref, bsum2_ref,
              h_ref, d1T_ref, out_ref, hT_s):
    i = pl.program_id(0)
    hT_s[pl.ds(i * RB1, RB1), :] = jax.nn.relu(
        jnp.dot(wfc1_ref[...], xpT_ref[...],
                preferred_element_type=jnp.float32)
        + bfc1_ref[...])

    @pl.when(i == NR1 - 1)
    def _():
        hT = hT_s[...]
        d1 = jax.nn.relu(jnp.dot(wfc2_ref[...], hT,
                                 preferred_element_type=jnp.float32)
                         + bfc2_ref[...])
        cat = jnp.concatenate([hT, xn2_ref[...]], axis=0)
        lg = jnp.dot(wsum2_ref[...], cat,
                     preferred_element_type=jnp.float32) + bsum2_ref[...]
        m = jnp.max(lg, axis=0, keepdims=True)
        lse = jnp.log(jnp.sum(jnp.exp(lg - m), axis=0, keepdims=True)) + m
        h_ref[...] = hT.T
        d1T_ref[...] = d1
        out_ref[...] = (lg - lse).T


def _mid(xpT, xn2, W_fc1, b_fc1, W_fc2, b_fc2, W_sum2, b_sum2):
    full = lambda shape: pl.BlockSpec(shape, lambda i: (0, 0))
    return pl.pallas_call(
        _mid_body,
        grid=(NR1,),
        in_specs=[
            full((FC1Fin, B)),
            pl.BlockSpec((RB1, FC1Fin), lambda i: (i, 0)),
            full((128, B)),
            pl.BlockSpec((RB1, 1), lambda i: (i, 0)),
            full((512, 256)), full((512, 1)),
            full((2, 384)), full((2, 1)),
        ],
        out_specs=[full((B, 256)), full((512, B)), full((B, 2))],
        out_shape=[
            jax.ShapeDtypeStruct((B, 256), jnp.float32),
            jax.ShapeDtypeStruct((512, B), jnp.float32),
            jax.ShapeDtypeStruct((B, 2), jnp.float32),
        ],
        scratch_shapes=[pltpu.VMEM((256, B), jnp.float32)],
    )(xpT, W_fc1, xn2,
      b_fc1.reshape(256, 1), W_fc2, b_fc2.reshape(512, 1), W_sum2,
      b_sum2.reshape(2, 1))


MB3 = 1000


def _fc3_body(wfc3_ref, d1T_ref, b_ref, dec_ref):
    dec_ref[...] = (jnp.dot(wfc3_ref[...], d1T_ref[...],
                            preferred_element_type=jnp.float32) + b_ref[...])


def _fc3(W_fc3, d1T, b_fc3):
    n = V // MB3
    return pl.pallas_call(
        _fc3_body,
        grid=(n,),
        in_specs=[
            pl.BlockSpec((MB3, 512), lambda i: (i, 0)),
            pl.BlockSpec((512, B), lambda i: (0, 0)),
            pl.BlockSpec((MB3, 1), lambda i: (i, 0)),
        ],
        out_specs=pl.BlockSpec((MB3, B), lambda i: (i, 0)),
        out_shape=jax.ShapeDtypeStruct((V, B), jnp.float32),
    )(W_fc3, d1T, b_fc3.reshape(V, 1))


def kernel(x_in, d, L_indices, L_values, W_cl1, b_cl1, W_fc1, b_fc1,
           W_fc2, b_fc2, W_fc3, b_fc3, W_nn1, b_nn1, W_nn2, b_nn2,
           W_sum2, b_sum2):
    x0 = jnp.transpose(x_in)
    x0p = jnp.pad(x0, ((0, V_PAD - V), (0, 0)))

    npad = EPAD - E
    pad_idx = (jnp.arange(npad, dtype=jnp.int32) % V)
    rows = jnp.concatenate([L_indices[0], pad_idx])
    cols = jnp.concatenate([L_indices[1], pad_idx])
    vals = jnp.concatenate([L_values, jnp.zeros((npad,), jnp.float32)])
    rows2 = rows.reshape(EPAD // IDXW, IDXW)
    cols2 = cols.reshape(EPAD // IDXW, IDXW)

    x1, x2, x3 = _spmm3(x0p, cols2, rows2, vals)
    xpT = _head(x0p, x1, x2, x3, W_cl1, b_cl1)[:FC1Fin]
    xn2 = _nn(x0, W_nn1, b_nn1, W_nn2, b_nn2)
    h, d1T, out = _mid(xpT, xn2, W_fc1, b_fc1, W_fc2, b_fc2, W_sum2, b_sum2)
    decT = _fc3(W_fc3, d1T, b_fc3)
    return (jnp.transpose(decT), h, out)


# speedup vs baseline: 14.2581x; 1.6401x over previous
import functools

import jax
import jax.numpy as jnp
from jax import lax
from jax.experimental import pallas as pl
from jax.experimental.pallas import tpu as pltpu
from jax.experimental.pallas import tpu_sc as plsc

B = 16
V = 10000
E = 320000
CL1_F = 32
CL1_K = 4
POOL = 8
FC1Fin = CL1_F * (V // POOL)
V_PAD = 10240

NCORES = 2
NTILES = 16
IDXW = 128
E2ROWS = 2560
EPAD = E2ROWS * IDXW
ROWS_PER_CT = E2ROWS // (NCORES * NTILES)
CHUNK_ROWS = 8
CE = CHUNK_ROWS * IDXW
NPAIR = ROWS_PER_CT // (2 * CHUNK_ROWS)
VROWS_T = V_PAD // NTILES
VH = VROWS_T // 2


def _make_spmm_body(do_combine, first):
    def body(y_hbm, prev_hbm, pprev_hbm, cols_hbm, rows_hbm, vals_hbm,
             xk_hbm, p_hbm,
             acc_sh, colsbA, rowsbA, valsbA, gbufA, sbufA,
             colsbB, rowsbB, valsbB, gbufB, sbufB,
             p0v, p1v, yv, pv, obuf, zbuf, gsemA, gsemB, ssem):
        cid = lax.axis_index("c")
        tid = lax.axis_index("s")

        @plsc.parallel_loop(0, VH, unroll=8)
        def _zinit(i):
            zbuf[i] = jnp.zeros((16,), jnp.float32)

        r0 = tid * VROWS_T
        for h in range(2):
            pltpu.sync_copy(zbuf, acc_sh.at[pl.ds(r0 + h * VH, VH)])

        if do_combine:
            for h in range(2):
                rh = r0 + h * VH
                pltpu.sync_copy(prev_hbm.at[pl.ds(rh, VH)], p0v)
                pltpu.sync_copy(prev_hbm.at[pl.ds(V_PAD + rh, VH)], p1v)
                pltpu.sync_copy(y_hbm.at[pl.ds(rh, VH)], yv)
                if not first:
                    pltpu.sync_copy(pprev_hbm.at[pl.ds(rh, VH)], pv)

                @plsc.parallel_loop(0, VH, unroll=8)
                def _comb(i):
                    s = p0v[i] + p1v[i]
                    if first:
                        obuf[i] = s - yv[i]
                    else:
                        obuf[i] = 2.0 * s - 2.0 * yv[i] - pv[i]
                pltpu.sync_copy(obuf, xk_hbm.at[pl.ds(rh, VH)])
            src_hbm = xk_hbm
        else:
            src_hbm = y_hbm
        plsc.subcore_barrier()

        slots = ((colsbA, rowsbA, valsbA, gbufA, sbufA, gsemA),
                 (colsbB, rowsbB, valsbB, gbufB, sbufB, gsemB))

        def pair_body(p, _):
            gdescs = []
            for b2, (cb, rb, vb, gb, sb, gs) in enumerate(slots):
                base_row = ((cid * NTILES + tid) * ROWS_PER_CT
                            + (2 * p + b2) * CHUNK_ROWS)
                pltpu.sync_copy(cols_hbm.at[pl.ds(base_row, CHUNK_ROWS)], cb)
                pltpu.sync_copy(rows_hbm.at[pl.ds(base_row, CHUNK_ROWS)], rb)
                pltpu.sync_copy(vals_hbm.at[pl.ds(base_row * IDXW, CE)], vb)
                gdescs.append([pltpu.async_copy(
                    src_hbm.at[cb.at[j]],
                    gb.at[pl.ds(j * IDXW, IDXW)], gs)
                    for j in range(CHUNK_ROWS)])
            sdescs = []
            for b2, (cb, rb, vb, gb, sb, gs) in enumerate(slots):
                for d in gdescs[b2]:
                    d.wait()

                @plsc.parallel_loop(0, CE, step=16, unroll=2)
                def _scale(g, vb=vb, gb=gb, sb=sb):
                    for j in range(16):
                        sv = plsc.load_gather(
                            vb, [jnp.full((16,), g + j, jnp.int32)])
                        sb[g + j] = sv * gb[g + j]
                for j in range(CHUNK_ROWS):
                    sdescs.append(pltpu.async_copy(
                        sb.at[pl.ds(j * IDXW, IDXW)],
                        acc_sh.at[rb.at[j]], ssem, add=True))
            for d in sdescs:
                d.wait()
            return _
        lax.fori_loop(0, NPAIR, pair_body, 0)
        plsc.subcore_barrier()

        for h in range(2):
            rh = r0 + h * VH
            pltpu.sync_copy(acc_sh.at[pl.ds(rh, VH)], obuf)
            pltpu.sync_copy(obuf, p_hbm.at[pl.ds(cid * V_PAD + rh, VH)])
    return body


def _sc_call(do_combine, first, y, prev_p, pprev, cols2, rows2, vals_pad):
    out_type = [jax.ShapeDtypeStruct((V_PAD, B), jnp.float32),
                jax.ShapeDtypeStruct((NCORES * V_PAD, B), jnp.float32)]
    kern = functools.partial(
        pl.kernel,
        mesh=plsc.VectorSubcoreMesh(core_axis_name="c", subcore_axis_name="s"),
        compiler_params=pltpu.CompilerParams(needs_layout_passes=False,
                                             use_tc_tiling_on_sc=False),
        out_type=out_type,
        scratch_types=[
            pltpu.VMEM_SHARED((V_PAD, B), jnp.float32),
            pltpu.VMEM((CHUNK_ROWS, IDXW), jnp.int32),
            pltpu.VMEM((CHUNK_ROWS, IDXW), jnp.int32),
            pltpu.VMEM((CE,), jnp.float32),
            pltpu.VMEM((CE, B), jnp.float32),
            pltpu.VMEM((CE, B), jnp.float32),
            pltpu.VMEM((CHUNK_ROWS, IDXW), jnp.int32),
            pltpu.VMEM((CHUNK_ROWS, IDXW), jnp.int32),
            pltpu.VMEM((CE,), jnp.float32),
            pltpu.VMEM((CE, B), jnp.float32),
            pltpu.VMEM((CE, B), jnp.float32),
            pltpu.VMEM((VH, B), jnp.float32),
            pltpu.VMEM((VH, B), jnp.float32),
            pltpu.VMEM((VH, B), jnp.float32),
            pltpu.VMEM((VH, B), jnp.float32),
            pltpu.VMEM((VH, B), jnp.float32),
            pltpu.VMEM((VH, B), jnp.float32),
            pltpu.SemaphoreType.DMA,
            pltpu.SemaphoreType.DMA,
            pltpu.SemaphoreType.DMA,
        ],
    )(_make_spmm_body(do_combine, first))
    if not do_combine:
        return kern(y, y, y, cols2, rows2, vals_pad)
    return kern(y, prev_p, pprev, cols2, rows2, vals_pad)


def _spmm3(x0, cols2, rows2, vals_pad):
    _unused, p1 = _sc_call(False, False, x0, None, None, cols2, rows2,
                           vals_pad)
    x1, p2 = _sc_call(True, True, x0, p1, x0, cols2, rows2, vals_pad)
    x2, p3 = _sc_call(True, False, x1, p2, x0, cols2, rows2, vals_pad)
    return x1, x2, p3


VB1 = 1024


def _head_body(x0_ref, x1_ref, x2_ref, p30_ref, p31_ref, w_ref, b_ref,
               out_ref):
    x1 = x1_ref[...]
    x2 = x2_ref[...]
    x3 = 2.0 * (p30_ref[...] + p31_ref[...] - x2) - x1
    xcat = jnp.concatenate([x0_ref[...], x1, x2, x3], axis=1)
    wt = w_ref[...].T
    eye = jnp.eye(16, dtype=jnp.float32)
    w4 = (eye[None, :, :, None] * wt[:, None, None, :]).reshape(64, 512)
    bias = jnp.tile(b_ref[...].reshape(1, 32), (1, 16))
    xg = jax.nn.relu(jnp.dot(xcat, w4, preferred_element_type=jnp.float32)
                     + bias)
    xp = xg.reshape(VB1 // POOL, POOL, 512).max(axis=1)
    xp = xp.reshape(VB1 // POOL, 16, 32)
    xp = jnp.swapaxes(xp, 1, 2).reshape(VB1 // POOL * 32, 16)
    out_ref[...] = xp


def _head(x0, x1, x2, p3, W_cl1, b_cl1):
    n = V_PAD // VB1
    return pl.pallas_call(
        _head_body,
        grid=(n,),
        in_specs=[
            pl.BlockSpec((VB1, B), lambda i: (i, 0)),
            pl.BlockSpec((VB1, B), lambda i: (i, 0)),
            pl.BlockSpec((VB1, B), lambda i: (i, 0)),
            pl.BlockSpec((VB1, B), lambda i: (i, 0)),
            pl.BlockSpec((VB1, B), lambda i: (V_PAD // VB1 + i, 0)),
            pl.BlockSpec((CL1_F, CL1_K), lambda i: (0, 0)),
            pl.BlockSpec((CL1_F, 1), lambda i: (0, 0)),
        ],
        out_specs=pl.BlockSpec((VB1 // POOL * CL1_F, B), lambda i: (i, 0)),
        out_shape=jax.ShapeDtypeStruct((V_PAD // POOL * CL1_F, B), jnp.float32),
    )(x0, x1, x2, p3, p3, W_cl1, b_cl1.reshape(CL1_F, 1))


def _nn_body(x0_ref, wnn1_ref, bnn1_ref, wnn2_ref, bnn2_ref, xn2_ref):
    xn = jax.nn.relu(jnp.dot(wnn1_ref[...], x0_ref[...],
                             preferred_element_type=jnp.float32)
                     + bnn1_ref[...])
    xn2_ref[...] = jax.nn.relu(jnp.dot(wnn2_ref[...], xn,
                                       preferred_element_type=jnp.float32)
                               + bnn2_ref[...])


def _nn(x0, W_nn1, b_nn1, W_nn2, b_nn2):
    full = lambda shape: pl.BlockSpec(shape, lambda: (0, 0))
    return pl.pallas_call(
        _nn_body,
        in_specs=[full((V, B)), full((256, V)), full((256, 1)),
                  full((128, 256)), full((128, 1))],
        out_specs=full((128, B)),
        out_shape=jax.ShapeDtypeStruct((128, B), jnp.float32),
    )(x0, W_nn1, b_nn1.reshape(256, 1), W_nn2, b_nn2.reshape(128, 1))


RB1 = 32
NR1 = 256 // RB1


def _mid_body(xpT_ref, wfc1_ref, xn2_ref,
              bfc1_ref, wfc2_ref, bfc2_ref, wsum2_ref, bsum2_ref,
              h_ref, d1T_ref, out_ref, hT_s):
    i = pl.program_id(0)
    hT_s[pl.ds(i * RB1, RB1), :] = jax.nn.relu(
        jnp.dot(wfc1_ref[...], xpT_ref[...],
                preferred_element_type=jnp.float32)
        + bfc1_ref[...])

    @pl.when(i == NR1 - 1)
    def _():
        hT = hT_s[...]
        d1 = jax.nn.relu(jnp.dot(wfc2_ref[...], hT,
                                 preferred_element_type=jnp.float32)
                         + bfc2_ref[...])
        cat = jnp.concatenate([hT, xn2_ref[...]], axis=0)
        lg = jnp.dot(wsum2_ref[...], cat,
                     preferred_element_type=jnp.float32) + bsum2_ref[...]
        m = jnp.max(lg, axis=0, keepdims=True)
        lse = jnp.log(jnp.sum(jnp.exp(lg - m), axis=0, keepdims=True)) + m
        h_ref[...] = hT.T
        d1T_ref[...] = d1
        out_ref[...] = (lg - lse).T


def _mid(xpT, xn2, W_fc1, b_fc1, W_fc2, b_fc2, W_sum2, b_sum2):
    full = lambda shape: pl.BlockSpec(shape, lambda i: (0, 0))
    return pl.pallas_call(
        _mid_body,
        grid=(NR1,),
        in_specs=[
            full((FC1Fin, B)),
            pl.BlockSpec((RB1, FC1Fin), lambda i: (i, 0)),
            full((128, B)),
            pl.BlockSpec((RB1, 1), lambda i: (i, 0)),
            full((512, 256)), full((512, 1)),
            full((2, 384)), full((2, 1)),
        ],
        out_specs=[full((B, 256)), full((512, B)), full((B, 2))],
        out_shape=[
            jax.ShapeDtypeStruct((B, 256), jnp.float32),
            jax.ShapeDtypeStruct((512, B), jnp.float32),
            jax.ShapeDtypeStruct((B, 2), jnp.float32),
        ],
        scratch_shapes=[pltpu.VMEM((256, B), jnp.float32)],
    )(xpT, W_fc1, xn2,
      b_fc1.reshape(256, 1), W_fc2, b_fc2.reshape(512, 1), W_sum2,
      b_sum2.reshape(2, 1))


MB3 = 1000


def _fc3_body(wfc3_ref, d1T_ref, b_ref, dec_ref):
    dec_ref[...] = (jnp.dot(wfc3_ref[...], d1T_ref[...],
                            preferred_element_type=jnp.float32) + b_ref[...])


def _fc3(W_fc3, d1T, b_fc3):
    n = V // MB3
    return pl.pallas_call(
        _fc3_body,
        grid=(n,),
        in_specs=[
            pl.BlockSpec((MB3, 512), lambda i: (i, 0)),
            pl.BlockSpec((512, B), lambda i: (0, 0)),
            pl.BlockSpec((MB3, 1), lambda i: (i, 0)),
        ],
        out_specs=pl.BlockSpec((MB3, B), lambda i: (i, 0)),
        out_shape=jax.ShapeDtypeStruct((V, B), jnp.float32),
    )(W_fc3, d1T, b_fc3.reshape(V, 1))


def kernel(x_in, d, L_indices, L_values, W_cl1, b_cl1, W_fc1, b_fc1,
           W_fc2, b_fc2, W_fc3, b_fc3, W_nn1, b_nn1, W_nn2, b_nn2,
           W_sum2, b_sum2):
    x0 = jnp.transpose(x_in)
    x0p = jnp.pad(x0, ((0, V_PAD - V), (0, 0)))

    npad = EPAD - E
    pad_idx = (jnp.arange(npad, dtype=jnp.int32) % V)
    rows = jnp.concatenate([L_indices[0], pad_idx])
    cols = jnp.concatenate([L_indices[1], pad_idx])
    vals = jnp.concatenate([L_values, jnp.zeros((npad,), jnp.float32)])
    rows2 = rows.reshape(EPAD // IDXW, IDXW)
    cols2 = cols.reshape(EPAD // IDXW, IDXW)

    x1, x2, p3 = _spmm3(x0p, cols2, rows2, vals)
    xpT = _head(x0p, x1, x2, p3, W_cl1, b_cl1)[:FC1Fin]
    xn2 = _nn(x0, W_nn1, b_nn1, W_nn2, b_nn2)
    h, d1T, out = _mid(xpT, xn2, W_fc1, b_fc1, W_fc2, b_fc2, W_sum2, b_sum2)
    decT = _fc3(W_fc3, d1T, b_fc3)
    return (jnp.transpose(decT), h, out)


# speedup vs baseline: 15.5296x; 1.0892x over previous
import functools

import jax
import jax.numpy as jnp
from jax import lax
from jax.experimental import pallas as pl
from jax.experimental.pallas import tpu as pltpu
from jax.experimental.pallas import tpu_sc as plsc

B = 16
V = 10000
E = 320000
CL1_F = 32
CL1_K = 4
POOL = 8
FC1Fin = CL1_F * (V // POOL)
V_PAD = 10240

NCORES = 2
NTILES = 16
IDXW = 128
E2ROWS = 2560
EPAD = E2ROWS * IDXW
ROWS_PER_CT = E2ROWS // (NCORES * NTILES)
CHUNK_ROWS = 8
CE = CHUNK_ROWS * IDXW
NPAIR = ROWS_PER_CT // (2 * CHUNK_ROWS)
VROWS_T = V_PAD // NTILES
VH = VROWS_T // 2


def _make_spmm_body(do_combine, first):
    def body(y_hbm, prev_hbm, pprev_hbm, cols_hbm, rows_hbm, vals_hbm,
             xk_hbm, p_hbm,
             acc_sh,
             colsbA, rowsbA, valsbA, gbufA,
             colsbB, rowsbB, valsbB, gbufB,
             colsbC, rowsbC, valsbC, gbufC,
             p0v, p1v, yv, pv, obuf, zbuf,
             isemA, gsemA, ssemA, isemB, gsemB, ssemB, isemC, gsemC, ssemC):
        cid = lax.axis_index("c")
        tid = lax.axis_index("s")

        @plsc.parallel_loop(0, VH, unroll=8)
        def _zinit(i):
            zbuf[i] = jnp.zeros((16,), jnp.float32)

        r0 = tid * VROWS_T
        for h in range(2):
            pltpu.sync_copy(zbuf, acc_sh.at[pl.ds(r0 + h * VH, VH)])

        if do_combine:
            for h in range(2):
                rh = r0 + h * VH
                pltpu.sync_copy(prev_hbm.at[pl.ds(rh, VH)], p0v)
                pltpu.sync_copy(prev_hbm.at[pl.ds(V_PAD + rh, VH)], p1v)
                pltpu.sync_copy(y_hbm.at[pl.ds(rh, VH)], yv)
                if not first:
                    pltpu.sync_copy(pprev_hbm.at[pl.ds(rh, VH)], pv)

                @plsc.parallel_loop(0, VH, unroll=8)
                def _comb(i):
                    s = p0v[i] + p1v[i]
                    if first:
                        obuf[i] = s - yv[i]
                    else:
                        obuf[i] = 2.0 * s - 2.0 * yv[i] - pv[i]
                pltpu.sync_copy(obuf, xk_hbm.at[pl.ds(rh, VH)])
            src_hbm = xk_hbm
        else:
            src_hbm = y_hbm
        plsc.subcore_barrier()

        slots = ((colsbA, rowsbA, valsbA, gbufA, isemA, gsemA, ssemA),
                 (colsbB, rowsbB, valsbB, gbufB, isemB, gsemB, ssemB),
                 (colsbC, rowsbC, valsbC, gbufC, isemC, gsemC, ssemC))
        NC_CT = ROWS_PER_CT // CHUNK_ROWS

        def fire_idx(c):
            cb, rb, vb, gb, isem, gs, ss = slots[c % 3]
            base_row = (cid * NTILES + tid) * ROWS_PER_CT + c * CHUNK_ROWS
            return [
                pltpu.async_copy(cols_hbm.at[pl.ds(base_row, CHUNK_ROWS)],
                                 cb, isem),
                pltpu.async_copy(rows_hbm.at[pl.ds(base_row, CHUNK_ROWS)],
                                 rb, isem),
                pltpu.async_copy(vals_hbm.at[pl.ds(base_row * IDXW, CE)],
                                 vb, isem),
            ]

        def fire_g(c):
            cb, rb, vb, gb, isem, gs, ss = slots[c % 3]
            return [pltpu.async_copy(src_hbm.at[cb.at[j]],
                                     gb.at[pl.ds(j * IDXW, IDXW)], gs)
                    for j in range(CHUNK_ROWS)]

        def fire_s(c):
            cb, rb, vb, gb, isem, gs, ss = slots[c % 3]
            return [pltpu.async_copy(gb.at[pl.ds(j * IDXW, IDXW)],
                                     acc_sh.at[rb.at[j]], ss, add=True)
                    for j in range(CHUNK_ROWS)]

        idescs = {0: fire_idx(0), 1: fire_idx(1)}
        gdescs = {}
        sdescs = {}
        for d in idescs.pop(0):
            d.wait()
        gdescs[0] = fire_g(0)
        for c in range(NC_CT):
            cb, rb, vb, gb, isem, gs, ss = slots[c % 3]
            if c + 2 < NC_CT:
                if c - 1 >= 0:
                    for d in sdescs.pop(c - 1):
                        d.wait()
                idescs[c + 2] = fire_idx(c + 2)
            if c + 1 < NC_CT:
                for d in idescs.pop(c + 1):
                    d.wait()
                gdescs[c + 1] = fire_g(c + 1)
            for d in gdescs.pop(c):
                d.wait()

            @plsc.parallel_loop(0, CE, step=16, unroll=2)
            def _scale(g, vb=vb, gb=gb):
                for j in range(16):
                    sv = plsc.load_gather(
                        vb, [jnp.full((16,), g + j, jnp.int32)])
                    gb[g + j] = sv * gb[g + j]
            sdescs[c] = fire_s(c)
        for c in (NC_CT - 2, NC_CT - 1):
            for d in sdescs.pop(c):
                d.wait()
        plsc.subcore_barrier()

        for h in range(2):
            rh = r0 + h * VH
            pltpu.sync_copy(acc_sh.at[pl.ds(rh, VH)], obuf)
            pltpu.sync_copy(obuf, p_hbm.at[pl.ds(cid * V_PAD + rh, VH)])
    return body


def _sc_call(do_combine, first, y, prev_p, pprev, cols2, rows2, vals_pad):
    out_type = [jax.ShapeDtypeStruct((V_PAD, B), jnp.float32),
                jax.ShapeDtypeStruct((NCORES * V_PAD, B), jnp.float32)]
    kern = functools.partial(
        pl.kernel,
        mesh=plsc.VectorSubcoreMesh(core_axis_name="c", subcore_axis_name="s"),
        compiler_params=pltpu.CompilerParams(needs_layout_passes=False,
                                             use_tc_tiling_on_sc=False),
        out_type=out_type,
        scratch_types=[
            pltpu.VMEM_SHARED((V_PAD, B), jnp.float32),
        ] + [
            t for _ in range(3) for t in (
                pltpu.VMEM((CHUNK_ROWS, IDXW), jnp.int32),
                pltpu.VMEM((CHUNK_ROWS, IDXW), jnp.int32),
                pltpu.VMEM((CE,), jnp.float32),
                pltpu.VMEM((CE, B), jnp.float32),
            )
        ] + [
            pltpu.VMEM((VH, B), jnp.float32),
            pltpu.VMEM((VH, B), jnp.float32),
            pltpu.VMEM((VH, B), jnp.float32),
            pltpu.VMEM((VH, B), jnp.float32),
            pltpu.VMEM((VH, B), jnp.float32),
            pltpu.VMEM((VH, B), jnp.float32),
        ] + [pltpu.SemaphoreType.DMA] * 9,
    )(_make_spmm_body(do_combine, first))
    if not do_combine:
        return kern(y, y, y, cols2, rows2, vals_pad)
    return kern(y, prev_p, pprev, cols2, rows2, vals_pad)


def _spmm3(x0, cols2, rows2, vals_pad):
    _unused, p1 = _sc_call(False, False, x0, None, None, cols2, rows2,
                           vals_pad)
    x1, p2 = _sc_call(True, True, x0, p1, x0, cols2, rows2, vals_pad)
    x2, p3 = _sc_call(True, False, x1, p2, x0, cols2, rows2, vals_pad)
    return x1, x2, p3


VB1 = 1024


def _head_body(x0_ref, x1_ref, x2_ref, p30_ref, p31_ref, w_ref, b_ref,
               out_ref):
    x1 = x1_ref[...]
    x2 = x2_ref[...]
    x3 = 2.0 * (p30_ref[...] + p31_ref[...] - x2) - x1
    xcat = jnp.concatenate([x0_ref[...], x1, x2, x3], axis=1)
    wt = w_ref[...].T
    eye = jnp.eye(16, dtype=jnp.float32)
    w4 = (eye[None, :, :, None] * wt[:, None, None, :]).reshape(64, 512)
    bias = jnp.tile(b_ref[...].reshape(1, 32), (1, 16))
    xg = jax.nn.relu(jnp.dot(xcat, w4, preferred_element_type=jnp.float32)
                     + bias)
    xp = xg.reshape(VB1 // POOL, POOL, 512).max(axis=1)
    xp = xp.reshape(VB1 // POOL, 16, 32)
    xp = jnp.swapaxes(xp, 1, 2).reshape(VB1 // POOL * 32, 16)
    out_ref[...] = xp


def _head(x0, x1, x2, p3, W_cl1, b_cl1):
    n = V_PAD // VB1
    return pl.pallas_call(
        _head_body,
        grid=(n,),
        in_specs=[
            pl.BlockSpec((VB1, B), lambda i: (i, 0)),
            pl.BlockSpec((VB1, B), lambda i: (i, 0)),
            pl.BlockSpec((VB1, B), lambda i: (i, 0)),
            pl.BlockSpec((VB1, B), lambda i: (i, 0)),
            pl.BlockSpec((VB1, B), lambda i: (V_PAD // VB1 + i, 0)),
            pl.BlockSpec((CL1_F, CL1_K), lambda i: (0, 0)),
            pl.BlockSpec((CL1_F, 1), lambda i: (0, 0)),
        ],
        out_specs=pl.BlockSpec((VB1 // POOL * CL1_F, B), lambda i: (i, 0)),
        out_shape=jax.ShapeDtypeStruct((V_PAD // POOL * CL1_F, B), jnp.float32),
    )(x0, x1, x2, p3, p3, W_cl1, b_cl1.reshape(CL1_F, 1))


def _nn_body(x0_ref, wnn1_ref, bnn1_ref, wnn2_ref, bnn2_ref, xn2_ref):
    xn = jax.nn.relu(jnp.dot(wnn1_ref[...], x0_ref[...],
                             preferred_element_type=jnp.float32)
                     + bnn1_ref[...])
    xn2_ref[...] = jax.nn.relu(jnp.dot(wnn2_ref[...], xn,
                                       preferred_element_type=jnp.float32)
                               + bnn2_ref[...])


def _nn(x0, W_nn1, b_nn1, W_nn2, b_nn2):
    full = lambda shape: pl.BlockSpec(shape, lambda: (0, 0))
    return pl.pallas_call(
        _nn_body,
        in_specs=[full((V, B)), full((256, V)), full((256, 1)),
                  full((128, 256)), full((128, 1))],
        out_specs=full((128, B)),
        out_shape=jax.ShapeDtypeStruct((128, B), jnp.float32),
    )(x0, W_nn1, b_nn1.reshape(256, 1), W_nn2, b_nn2.reshape(128, 1))


RB1 = 32
NR1 = 256 // RB1


def _mid_body(xpT_ref, wfc1_ref, xn2_ref,
              bfc1_ref, wfc2_ref, bfc2_ref, wsum2_ref, bsum2_ref,
              h_ref, d1T_ref, out_ref, hT_s):
    i = pl.program_id(0)
    hT_s[pl.ds(i * RB1, RB1), :] = jax.nn.relu(
        jnp.dot(wfc1_ref[...], xpT_ref[...],
                preferred_element_type=jnp.float32)
        + bfc1_ref[...])

    @pl.when(i == NR1 - 1)
    def _():
        hT = hT_s[...]
        d1 = jax.nn.relu(jnp.dot(wfc2_ref[...], hT,
                                 preferred_element_type=jnp.float32)
                         + bfc2_ref[...])
        cat = jnp.concatenate([hT, xn2_ref[...]], axis=0)
        lg = jnp.dot(wsum2_ref[...], cat,
                     preferred_element_type=jnp.float32) + bsum2_ref[...]
        m = jnp.max(lg, axis=0, keepdims=True)
        lse = jnp.log(jnp.sum(jnp.exp(lg - m), axis=0, keepdims=True)) + m
        h_ref[...] = hT.T
        d1T_ref[...] = d1
        out_ref[...] = (lg - lse).T


def _mid(xpT, xn2, W_fc1, b_fc1, W_fc2, b_fc2, W_sum2, b_sum2):
    full = lambda shape: pl.BlockSpec(shape, lambda i: (0, 0))
    return pl.pallas_call(
        _mid_body,
        grid=(NR1,),
        in_specs=[
            pl.BlockSpec((FC1Fin, B), lambda i: (0, 0)),
            pl.BlockSpec((RB1, FC1Fin), lambda i: (i, 0)),
            full((128, B)),
            pl.BlockSpec((RB1, 1), lambda i: (i, 0)),
            full((512, 256)), full((512, 1)),
            full((2, 384)), full((2, 1)),
        ],
        out_specs=[full((B, 256)), full((512, B)), full((B, 2))],
        out_shape=[
            jax.ShapeDtypeStruct((B, 256), jnp.float32),
            jax.ShapeDtypeStruct((512, B), jnp.float32),
            jax.ShapeDtypeStruct((B, 2), jnp.float32),
        ],
        scratch_shapes=[pltpu.VMEM((256, B), jnp.float32)],
    )(xpT, W_fc1, xn2,
      b_fc1.reshape(256, 1), W_fc2, b_fc2.reshape(512, 1), W_sum2,
      b_sum2.reshape(2, 1))


MB3 = 1000


def _fc3_body(wfc3_ref, d1T_ref, b_ref, dec_ref):
    dec_ref[...] = (jnp.dot(wfc3_ref[...], d1T_ref[...],
                            preferred_element_type=jnp.float32) + b_ref[...])


def _fc3(W_fc3, d1T, b_fc3):
    n = V // MB3
    return pl.pallas_call(
        _fc3_body,
        grid=(n,),
        in_specs=[
            pl.BlockSpec((MB3, 512), lambda i: (i, 0)),
            pl.BlockSpec((512, B), lambda i: (0, 0)),
            pl.BlockSpec((MB3, 1), lambda i: (i, 0)),
        ],
        out_specs=pl.BlockSpec((MB3, B), lambda i: (i, 0)),
        out_shape=jax.ShapeDtypeStruct((V, B), jnp.float32),
    )(W_fc3, d1T, b_fc3.reshape(V, 1))


def kernel(x_in, d, L_indices, L_values, W_cl1, b_cl1, W_fc1, b_fc1,
           W_fc2, b_fc2, W_fc3, b_fc3, W_nn1, b_nn1, W_nn2, b_nn2,
           W_sum2, b_sum2):
    x0 = jnp.transpose(x_in)
    x0p = jnp.pad(x0, ((0, V_PAD - V), (0, 0)))

    npad = EPAD - E
    pad_idx = (jnp.arange(npad, dtype=jnp.int32) % V)
    rows = jnp.concatenate([L_indices[0], pad_idx])
    cols = jnp.concatenate([L_indices[1], pad_idx])
    vals = jnp.concatenate([L_values, jnp.zeros((npad,), jnp.float32)])
    rows2 = rows.reshape(EPAD // IDXW, IDXW)
    cols2 = cols.reshape(EPAD // IDXW, IDXW)

    xn2 = _nn(x0, W_nn1, b_nn1, W_nn2, b_nn2)
    x1, x2, p3 = _spmm3(x0p, cols2, rows2, vals)
    xpT = _head(x0p, x1, x2, p3, W_cl1, b_cl1)
    h, d1T, out = _mid(xpT, xn2, W_fc1, b_fc1, W_fc2, b_fc2, W_sum2, b_sum2)
    decT = _fc3(W_fc3, d1T, b_fc3)
    return (jnp.transpose(decT), h, out)


# speedup vs baseline: 15.5996x; 1.0045x over previous
import functools

import jax
import jax.numpy as jnp
from jax import lax
from jax.experimental import pallas as pl
from jax.experimental.pallas import tpu as pltpu
from jax.experimental.pallas import tpu_sc as plsc

B = 16
V = 10000
E = 320000
CL1_F = 32
CL1_K = 4
POOL = 8
FC1Fin = CL1_F * (V // POOL)
V_PAD = 10240

NCORES = 2
NTILES = 16
IDXW = 128
E2ROWS = 2560
EPAD = E2ROWS * IDXW
ROWS_PER_CT = E2ROWS // (NCORES * NTILES)
CHUNK_ROWS = 8
CE = CHUNK_ROWS * IDXW
NPAIR = ROWS_PER_CT // (2 * CHUNK_ROWS)
VROWS_T = V_PAD // NTILES
VH = VROWS_T // 2


def _make_spmm_body(do_combine, first):
    def body(y_hbm, prev_hbm, pprev_hbm, cols_hbm, rows_hbm, vals_hbm,
             xk_hbm, p_hbm,
             acc_sh,
             colsbA, rowsbA, valsbA, gbufA,
             colsbB, rowsbB, valsbB, gbufB,
             colsbC, rowsbC, valsbC, gbufC,
             p0v, p1v, yv, pv, obuf, zbuf,
             isemA, gsemA, ssemA, isemB, gsemB, ssemB, isemC, gsemC, ssemC):
        cid = lax.axis_index("c")
        tid = lax.axis_index("s")

        @plsc.parallel_loop(0, VH, unroll=8)
        def _zinit(i):
            zbuf[i] = jnp.zeros((16,), jnp.float32)

        r0 = tid * VROWS_T
        for h in range(2):
            pltpu.sync_copy(zbuf, acc_sh.at[pl.ds(r0 + h * VH, VH)])

        if do_combine:
            for h in range(2):
                rh = r0 + h * VH
                pltpu.sync_copy(prev_hbm.at[pl.ds(rh, VH)], p0v)
                pltpu.sync_copy(prev_hbm.at[pl.ds(V_PAD + rh, VH)], p1v)
                pltpu.sync_copy(y_hbm.at[pl.ds(rh, VH)], yv)
                if not first:
                    pltpu.sync_copy(pprev_hbm.at[pl.ds(rh, VH)], pv)

                @plsc.parallel_loop(0, VH, unroll=8)
                def _comb(i):
                    s = p0v[i] + p1v[i]
                    if first:
                        obuf[i] = s - yv[i]
                    else:
                        obuf[i] = 2.0 * s - 2.0 * yv[i] - pv[i]
                pltpu.sync_copy(obuf, xk_hbm.at[pl.ds(rh, VH)])
            src_hbm = xk_hbm
        else:
            src_hbm = y_hbm
        plsc.subcore_barrier()

        slots = ((colsbA, rowsbA, valsbA, gbufA, isemA, gsemA, ssemA),
                 (colsbB, rowsbB, valsbB, gbufB, isemB, gsemB, ssemB),
                 (colsbC, rowsbC, valsbC, gbufC, isemC, gsemC, ssemC))
        NC_CT = ROWS_PER_CT // CHUNK_ROWS

        def fire_idx(c):
            cb, rb, vb, gb, isem, gs, ss = slots[c % 3]
            base_row = (cid * NTILES + tid) * ROWS_PER_CT + c * CHUNK_ROWS
            return [
                pltpu.async_copy(cols_hbm.at[pl.ds(base_row, CHUNK_ROWS)],
                                 cb, isem),
                pltpu.async_copy(rows_hbm.at[pl.ds(base_row, CHUNK_ROWS)],
                                 rb, isem),
                pltpu.async_copy(vals_hbm.at[pl.ds(base_row * IDXW, CE)],
                                 vb, isem),
            ]

        def fire_g(c):
            cb, rb, vb, gb, isem, gs, ss = slots[c % 3]
            return [pltpu.async_copy(src_hbm.at[cb.at[j]],
                                     gb.at[pl.ds(j * IDXW, IDXW)], gs)
                    for j in range(CHUNK_ROWS)]

        def fire_s(c):
            cb, rb, vb, gb, isem, gs, ss = slots[c % 3]
            return [pltpu.async_copy(gb.at[pl.ds(j * IDXW, IDXW)],
                                     acc_sh.at[rb.at[j]], ss, add=True)
                    for j in range(CHUNK_ROWS)]

        idescs = {0: fire_idx(0), 1: fire_idx(1)}
        gdescs = {}
        sdescs = {}
        for d in idescs.pop(0):
            d.wait()
        gdescs[0] = fire_g(0)
        for c in range(NC_CT):
            cb, rb, vb, gb, isem, gs, ss = slots[c % 3]
            if c + 2 < NC_CT:
                if c - 1 >= 0:
                    for d in sdescs.pop(c - 1):
                        d.wait()
                idescs[c + 2] = fire_idx(c + 2)
            if c + 1 < NC_CT:
                for d in idescs.pop(c + 1):
                    d.wait()
                gdescs[c + 1] = fire_g(c + 1)
            for d in gdescs.pop(c):
                d.wait()

            @plsc.parallel_loop(0, CE, step=16, unroll=2)
            def _scale(g, vb=vb, gb=gb):
                for j in range(16):
                    sv = plsc.load_gather(
                        vb, [jnp.full((16,), g + j, jnp.int32)])
                    gb[g + j] = sv * gb[g + j]
            sdescs[c] = fire_s(c)
        for c in (NC_CT - 3, NC_CT - 2, NC_CT - 1):
            for d in sdescs.pop(c):
                d.wait()
        plsc.subcore_barrier()

        for h in range(2):
            rh = r0 + h * VH
            pltpu.sync_copy(acc_sh.at[pl.ds(rh, VH)], obuf)
            pltpu.sync_copy(obuf, p_hbm.at[pl.ds(cid * V_PAD + rh, VH)])
    return body


def _sc_call(do_combine, first, y, prev_p, pprev, cols2, rows2, vals_pad):
    out_type = [jax.ShapeDtypeStruct((V_PAD, B), jnp.float32),
                jax.ShapeDtypeStruct((NCORES * V_PAD, B), jnp.float32)]
    kern = functools.partial(
        pl.kernel,
        mesh=plsc.VectorSubcoreMesh(core_axis_name="c", subcore_axis_name="s"),
        compiler_params=pltpu.CompilerParams(needs_layout_passes=False,
                                             use_tc_tiling_on_sc=False),
        out_type=out_type,
        scratch_types=[
            pltpu.VMEM_SHARED((V_PAD, B), jnp.float32),
        ] + [
            t for _ in range(3) for t in (
                pltpu.VMEM((CHUNK_ROWS, IDXW), jnp.int32),
                pltpu.VMEM((CHUNK_ROWS, IDXW), jnp.int32),
                pltpu.VMEM((CE,), jnp.float32),
                pltpu.VMEM((CE, B), jnp.float32),
            )
        ] + [
            pltpu.VMEM((VH, B), jnp.float32),
            pltpu.VMEM((VH, B), jnp.float32),
            pltpu.VMEM((VH, B), jnp.float32),
            pltpu.VMEM((VH, B), jnp.float32),
            pltpu.VMEM((VH, B), jnp.float32),
            pltpu.VMEM((VH, B), jnp.float32),
        ] + [pltpu.SemaphoreType.DMA] * 9,
    )(_make_spmm_body(do_combine, first))
    if not do_combine:
        return kern(y, y, y, cols2, rows2, vals_pad)
    return kern(y, prev_p, pprev, cols2, rows2, vals_pad)


def _spmm3(x0, cols2, rows2, vals_pad):
    _unused, p1 = _sc_call(False, False, x0, None, None, cols2, rows2,
                           vals_pad)
    x1, p2 = _sc_call(True, True, x0, p1, x0, cols2, rows2, vals_pad)
    x2, p3 = _sc_call(True, False, x1, p2, x0, cols2, rows2, vals_pad)
    return x1, x2, p3


VB1 = 1024


def _head_body(x0_ref, x1_ref, x2_ref, p30_ref, p31_ref, w_ref, b_ref,
               out_ref):
    x1 = x1_ref[...]
    x2 = x2_ref[...]
    x3 = 2.0 * (p30_ref[...] + p31_ref[...] - x2) - x1
    xcat = jnp.concatenate([x0_ref[...], x1, x2, x3], axis=1)
    wt = w_ref[...].T
    eye = jnp.eye(16, dtype=jnp.float32)
    w4 = (eye[None, :, :, None] * wt[:, None, None, :]).reshape(64, 512)
    bias = jnp.tile(b_ref[...].reshape(1, 32), (1, 16))
    xg = jax.nn.relu(jnp.dot(xcat, w4, preferred_element_type=jnp.float32)
                     + bias)
    xp = xg.reshape(VB1 // POOL, POOL, 512).max(axis=1)
    xp = xp.reshape(VB1 // POOL, 16, 32)
    xp = jnp.swapaxes(xp, 1, 2).reshape(VB1 // POOL * 32, 16)
    out_ref[...] = xp


def _head(x0, x1, x2, p3, W_cl1, b_cl1):
    n = V_PAD // VB1
    return pl.pallas_call(
        _head_body,
        grid=(n,),
        in_specs=[
            pl.BlockSpec((VB1, B), lambda i: (i, 0)),
            pl.BlockSpec((VB1, B), lambda i: (i, 0)),
            pl.BlockSpec((VB1, B), lambda i: (i, 0)),
            pl.BlockSpec((VB1, B), lambda i: (i, 0)),
            pl.BlockSpec((VB1, B), lambda i: (V_PAD // VB1 + i, 0)),
            pl.BlockSpec((CL1_F, CL1_K), lambda i: (0, 0)),
            pl.BlockSpec((CL1_F, 1), lambda i: (0, 0)),
        ],
        out_specs=pl.BlockSpec((VB1 // POOL * CL1_F, B), lambda i: (i, 0)),
        out_shape=jax.ShapeDtypeStruct((V_PAD // POOL * CL1_F, B), jnp.float32),
    )(x0, x1, x2, p3, p3, W_cl1, b_cl1.reshape(CL1_F, 1))


def _nn_body(x0_ref, wnn1_ref, bnn1_ref, wnn2_ref, bnn2_ref, xn2_ref):
    xn = jax.nn.relu(jnp.dot(wnn1_ref[...], x0_ref[...],
                             preferred_element_type=jnp.float32)
                     + bnn1_ref[...])
    xn2_ref[...] = jax.nn.relu(jnp.dot(wnn2_ref[...], xn,
                                       preferred_element_type=jnp.float32)
                               + bnn2_ref[...])


def _nn(x0, W_nn1, b_nn1, W_nn2, b_nn2):
    full = lambda shape: pl.BlockSpec(shape, lambda: (0, 0))
    return pl.pallas_call(
        _nn_body,
        in_specs=[full((V, B)), full((256, V)), full((256, 1)),
                  full((128, 256)), full((128, 1))],
        out_specs=full((128, B)),
        out_shape=jax.ShapeDtypeStruct((128, B), jnp.float32),
    )(x0, W_nn1, b_nn1.reshape(256, 1), W_nn2, b_nn2.reshape(128, 1))


RB1 = 32
NR1 = 256 // RB1


def _mid_body(xpT_ref, wfc1_ref, xn2_ref,
              bfc1_ref, wfc2_ref, bfc2_ref, wsum2_ref, bsum2_ref,
              h_ref, d1T_ref, out_ref, hT_s):
    i = pl.program_id(0)
    hT_s[pl.ds(i * RB1, RB1), :] = jax.nn.relu(
        jnp.dot(wfc1_ref[...], xpT_ref[...],
                preferred_element_type=jnp.float32)
        + bfc1_ref[...])

    @pl.when(i == NR1 - 1)
    def _():
        hT = hT_s[...]
        d1 = jax.nn.relu(jnp.dot(wfc2_ref[...], hT,
                                 preferred_element_type=jnp.float32)
                         + bfc2_ref[...])
        cat = jnp.concatenate([hT, xn2_ref[...]], axis=0)
        lg = jnp.dot(wsum2_ref[...], cat,
                     preferred_element_type=jnp.float32) + bsum2_ref[...]
        m = jnp.max(lg, axis=0, keepdims=True)
        lse = jnp.log(jnp.sum(jnp.exp(lg - m), axis=0, keepdims=True)) + m
        h_ref[...] = hT.T
        d1T_ref[...] = d1
        out_ref[...] = (lg - lse).T


def _mid(xpT, xn2, W_fc1, b_fc1, W_fc2, b_fc2, W_sum2, b_sum2):
    full = lambda shape: pl.BlockSpec(shape, lambda i: (0, 0))
    return pl.pallas_call(
        _mid_body,
        grid=(NR1,),
        in_specs=[
            pl.BlockSpec((FC1Fin, B), lambda i: (0, 0)),
            pl.BlockSpec((RB1, FC1Fin), lambda i: (i, 0)),
            full((128, B)),
            pl.BlockSpec((RB1, 1), lambda i: (i, 0)),
            full((512, 256)), full((512, 1)),
            full((2, 384)), full((2, 1)),
        ],
        out_specs=[full((B, 256)), full((512, B)), full((B, 2))],
        out_shape=[
            jax.ShapeDtypeStruct((B, 256), jnp.float32),
            jax.ShapeDtypeStruct((512, B), jnp.float32),
            jax.ShapeDtypeStruct((B, 2), jnp.float32),
        ],
        scratch_shapes=[pltpu.VMEM((256, B), jnp.float32)],
    )(xpT, W_fc1, xn2,
      b_fc1.reshape(256, 1), W_fc2, b_fc2.reshape(512, 1), W_sum2,
      b_sum2.reshape(2, 1))


MB3 = 1000


def _fc3_body(wfc3_ref, d1T_ref, b_ref, dec_ref):
    dec_ref[...] = (jnp.dot(wfc3_ref[...], d1T_ref[...],
                            preferred_element_type=jnp.float32) + b_ref[...])


def _fc3(W_fc3, d1T, b_fc3):
    n = V // MB3
    return pl.pallas_call(
        _fc3_body,
        grid=(n,),
        in_specs=[
            pl.BlockSpec((MB3, 512), lambda i: (i, 0)),
            pl.BlockSpec((512, B), lambda i: (0, 0)),
            pl.BlockSpec((MB3, 1), lambda i: (i, 0)),
        ],
        out_specs=pl.BlockSpec((MB3, B), lambda i: (i, 0)),
        out_shape=jax.ShapeDtypeStruct((V, B), jnp.float32),
    )(W_fc3, d1T, b_fc3.reshape(V, 1))


def kernel(x_in, d, L_indices, L_values, W_cl1, b_cl1, W_fc1, b_fc1,
           W_fc2, b_fc2, W_fc3, b_fc3, W_nn1, b_nn1, W_nn2, b_nn2,
           W_sum2, b_sum2):
    x0 = jnp.transpose(x_in)
    x0p = jnp.pad(x0, ((0, V_PAD - V), (0, 0)))

    npad = EPAD - E
    pad_idx = (jnp.arange(npad, dtype=jnp.int32) % V)
    rows = jnp.concatenate([L_indices[0], pad_idx])
    cols = jnp.concatenate([L_indices[1], pad_idx])
    vals = jnp.concatenate([L_values, jnp.zeros((npad,), jnp.float32)])
    rows2 = rows.reshape(EPAD // IDXW, IDXW)
    cols2 = cols.reshape(EPAD // IDXW, IDXW)

    xn2 = _nn(x0, W_nn1, b_nn1, W_nn2, b_nn2)
    x1, x2, p3 = _spmm3(x0p, cols2, rows2, vals)
    xpT = _head(x0p, x1, x2, p3, W_cl1, b_cl1)
    h, d1T, out = _mid(xpT, xn2, W_fc1, b_fc1, W_fc2, b_fc2, W_sum2, b_sum2)
    decT = _fc3(W_fc3, d1T, b_fc3)
    return (jnp.transpose(decT), h, out)
